# Initial kernel scaffold; baseline (speedup 1.0000x reference)
import jax
import jax.numpy as jnp
from jax.experimental import pallas as pl


def kernel(x, positions, pos_table):
    raise NotImplementedError("write your pallas kernel here")


# speedup vs baseline: 1.0260x; 1.0260x over previous
import functools

import jax
import jax.numpy as jnp
from jax import lax
from jax.experimental import pallas as pl
from jax.experimental.pallas import tpu as pltpu
from jax.experimental.pallas import tpu_sc as plsc

L = 16


def kernel(x, positions, pos_table):
    B, S, D = x.shape
    N = B * S
    xf = x.reshape(N, D)
    posf = positions.reshape(N).astype(jnp.int32)

    NC, NS = 2, 16
    NW = NC * NS
    rows_per_w = N // NW
    R = 32
    n_chunks = rows_per_w // R

    mesh = plsc.VectorSubcoreMesh(core_axis_name="c", subcore_axis_name="s")

    @functools.partial(
        pl.kernel,
        mesh=mesh,
        out_type=jax.ShapeDtypeStruct((N, D), jnp.float32),
        scratch_types=[
            pltpu.VMEM((R,), jnp.int32),
            pltpu.VMEM((R, D), jnp.float32),
            pltpu.VMEM((R, D), jnp.float32),
            pltpu.SemaphoreType.DMA,
            pltpu.SemaphoreType.DMA,
        ],
    )
    def pe_add(x_hbm, pos_hbm, tab_hbm, out_hbm, idx_v, pe_v, x_v, gsem, xsem):
        wid = lax.axis_index("s") * NC + lax.axis_index("c")
        base = wid * rows_per_w

        @pl.loop(0, n_chunks)
        def _(ci):
            rbase = base + ci * R
            pltpu.sync_copy(pos_hbm.at[pl.ds(rbase, R)], idx_v)
            gcp = pltpu.async_copy(tab_hbm.at[idx_v], pe_v, gsem)
            xcp = pltpu.async_copy(x_hbm.at[pl.ds(rbase, R), :], x_v, xsem)
            gcp.wait()
            xcp.wait()

            @pl.loop(0, R)
            def _(r):
                for j in range(0, D, L):
                    x_v[r, pl.ds(j, L)] += pe_v[r, pl.ds(j, L)]

            pltpu.sync_copy(x_v, out_hbm.at[pl.ds(rbase, R), :])

    out = pe_add(xf, posf, pos_table)
    return out.reshape(B, S, D)


# speedup vs baseline: 1.8649x; 1.8176x over previous
import functools

import jax
import jax.numpy as jnp
from jax import lax
from jax.experimental import pallas as pl
from jax.experimental.pallas import tpu as pltpu
from jax.experimental.pallas import tpu_sc as plsc

L = 16


def kernel(x, positions, pos_table):
    B, S, D = x.shape
    N = B * S
    xf = x.reshape(N, D)
    posf = positions.reshape(N).astype(jnp.int32)

    NC, NS = 2, 16
    NW = NC * NS
    rows_per_w = N // NW
    R = 16
    n_chunks = rows_per_w // R
    assert n_chunks % 2 == 0

    mesh = plsc.VectorSubcoreMesh(core_axis_name="c", subcore_axis_name="s")

    @functools.partial(
        pl.kernel,
        mesh=mesh,
        out_type=jax.ShapeDtypeStruct((N, D), jnp.float32),
        scratch_types=[
            pltpu.VMEM((rows_per_w,), jnp.int32),
            [pltpu.VMEM((R, D), jnp.float32)] * 2,
            [pltpu.VMEM((R, D), jnp.float32)] * 2,
            [pltpu.VMEM((R, D), jnp.float32)] * 2,
            [pltpu.SemaphoreType.DMA] * 2,
            [pltpu.SemaphoreType.DMA] * 2,
            [pltpu.SemaphoreType.DMA] * 2,
        ],
    )
    def pe_add(x_hbm, pos_hbm, tab_hbm, out_hbm,
               idx_v, pe_s, x_s, o_s, gsem, xsem, osem):
        wid = lax.axis_index("s") * NC + lax.axis_index("c")
        base = wid * rows_per_w

        pltpu.sync_copy(pos_hbm.at[pl.ds(base, rows_per_w)], idx_v)

        def start_in(c, b):
            pltpu.async_copy(tab_hbm.at[idx_v.at[pl.ds(c * R, R)]],
                             pe_s[b], gsem[b])
            pltpu.async_copy(x_hbm.at[pl.ds(base + c * R, R), :],
                             x_s[b], xsem[b])

        def wait_in(b):
            pltpu.make_async_copy(tab_hbm.at[idx_v.at[pl.ds(0, R)]],
                                  pe_s[b], gsem[b]).wait()
            pltpu.make_async_copy(x_hbm.at[pl.ds(0, R), :],
                                  x_s[b], xsem[b]).wait()

        def wait_out(b):
            pltpu.make_async_copy(o_s[b], out_hbm.at[pl.ds(0, R), :],
                                  osem[b]).wait()

        start_in(0, 0)
        start_in(1, 1)

        @pl.loop(0, n_chunks, step=2)
        def _(ci):
            for b in (0, 1):
                c = ci + b
                wait_in(b)

                @pl.when(c >= 2)
                def _():
                    wait_out(b)

                @pl.loop(0, R)
                def _(r):
                    for j in range(0, D, L):
                        o_s[b][r, pl.ds(j, L)] = (
                            x_s[b][r, pl.ds(j, L)] + pe_s[b][r, pl.ds(j, L)])

                pltpu.async_copy(o_s[b], out_hbm.at[pl.ds(base + c * R, R), :],
                                 osem[b])

                @pl.when(c + 2 < n_chunks)
                def _():
                    start_in(c + 2, b)

        wait_out(0)
        wait_out(1)

    out = pe_add(xf, posf, pos_table)
    return out.reshape(B, S, D)
